# baseline (speedup 1.0000x reference)
import functools

import jax
import jax.numpy as jnp
from jax import lax
from jax.experimental import pallas as pl
from jax.experimental.pallas import tpu as pltpu
from jax.experimental.pallas import tpu_sc as plsc

BATCH = 32
SEQ = 2048
D = 128
NUM_WORKERS = 32
CHUNK = SEQ // NUM_WORKERS
VECS = D // 16


@functools.partial(
    pl.kernel,
    mesh=plsc.VectorSubcoreMesh(core_axis_name="c", subcore_axis_name="s"),
    out_type=jax.ShapeDtypeStruct((BATCH, SEQ, D), jnp.float32),
    scratch_types=[
        pltpu.VMEM((CHUNK,), jnp.int32),
        pltpu.VMEM((CHUNK, D), jnp.float32),
        pltpu.VMEM((CHUNK, D), jnp.float32),
        pltpu.SemaphoreType.DMA,
    ],
)
def _emb_kernel(idx_hbm, table_hbm, pos_hbm, out_hbm, idx_v, pos_v, rows_v, sem):
    c = lax.axis_index("c")
    s = lax.axis_index("s")
    wid = s * 2 + c
    base = wid * CHUNK

    pltpu.sync_copy(pos_hbm.at[pl.ds(base, CHUNK), :], pos_v)

    def batch_body(b, carry):
        off = pl.multiple_of(b * SEQ + base, CHUNK)
        pltpu.sync_copy(idx_hbm.at[pl.ds(off, CHUNK)], idx_v)
        pltpu.async_copy(table_hbm.at[idx_v], rows_v, sem).wait()

        def add_body(r, carry2):
            for cc in range(VECS):
                sl = pl.ds(cc * 16, 16)
                rows_v[r, sl] = rows_v[r, sl] + pos_v[r, sl]
            return carry2

        lax.fori_loop(0, CHUNK, add_body, 0)
        pltpu.sync_copy(rows_v, out_hbm.at[b, pl.ds(base, CHUNK), :])
        return carry

    lax.fori_loop(0, BATCH, batch_body, 0)


def kernel(input_ids, emb_table, pos_enc):
    out = _emb_kernel(input_ids.astype(jnp.int32).reshape(BATCH * SEQ),
                      emb_table, pos_enc.reshape(SEQ, D))
    return out


# speedup vs baseline: 1.6989x; 1.6989x over previous
import functools

import jax
import jax.numpy as jnp
from jax import lax
from jax.experimental import pallas as pl
from jax.experimental.pallas import tpu as pltpu
from jax.experimental.pallas import tpu_sc as plsc

BATCH = 32
SEQ = 2048
D = 128
NUM_WORKERS = 32
CHUNK = SEQ // NUM_WORKERS
VECS = D // 16


@functools.partial(
    pl.kernel,
    mesh=plsc.VectorSubcoreMesh(core_axis_name="c", subcore_axis_name="s"),
    out_type=jax.ShapeDtypeStruct((BATCH, SEQ, D), jnp.float32),
    scratch_types=[
        pltpu.VMEM((CHUNK,), jnp.int32),
        pltpu.VMEM((CHUNK,), jnp.int32),
        pltpu.VMEM((CHUNK, D), jnp.float32),
        pltpu.VMEM((CHUNK, D), jnp.float32),
        pltpu.VMEM((CHUNK, D), jnp.float32),
        pltpu.SemaphoreType.DMA,
        pltpu.SemaphoreType.DMA,
        pltpu.SemaphoreType.DMA,
        pltpu.SemaphoreType.DMA,
        pltpu.SemaphoreType.DMA,
        pltpu.SemaphoreType.DMA,
    ],
)
def _emb_kernel(idx_hbm, table_hbm, pos_hbm, out_hbm,
                idx0, idx1, pos_v, rows0, rows1,
                gsem0, gsem1, wsem0, wsem1, isem0, isem1):
    c = lax.axis_index("c")
    s = lax.axis_index("s")
    wid = s * 2 + c
    base = wid * CHUNK

    idx_v = (idx0, idx1)
    rows_v = (rows0, rows1)
    gsem = (gsem0, gsem1)
    wsem = (wsem0, wsem1)
    isem = (isem0, isem1)

    def idx_off(b):
        return pl.multiple_of(b * SEQ + base, CHUNK)

    pltpu.sync_copy(pos_hbm.at[pl.ds(base, CHUNK), :], pos_v)

    pltpu.sync_copy(idx_hbm.at[pl.ds(idx_off(0), CHUNK)], idx_v[0])
    pltpu.async_copy(table_hbm.at[idx_v[0]], rows_v[0], gsem[0])
    pltpu.async_copy(idx_hbm.at[pl.ds(idx_off(1), CHUNK)], idx_v[1], isem[1])

    def add_pos(rows):
        def add_body(r2, carry):
            for u in range(2):
                r = r2 * 2 + u
                for cc in range(VECS):
                    sl = pl.ds(cc * 16, 16)
                    plsc.addupdate(rows.at[r, sl], pos_v[r, sl])
            return carry
        lax.fori_loop(0, CHUNK // 2, add_body, 0)

    def step(b, p):
        np_ = 1 - p
        @pl.when(b + 1 < BATCH)
        def _():
            pltpu.make_async_copy(
                idx_hbm.at[pl.ds(idx_off(0), CHUNK)], idx_v[np_], isem[np_]
            ).wait()
            @pl.when(b >= 1)
            def _():
                pltpu.make_async_copy(
                    rows_v[np_], out_hbm.at[0, pl.ds(base, CHUNK), :], wsem[np_]
                ).wait()
            pltpu.async_copy(table_hbm.at[idx_v[np_]], rows_v[np_], gsem[np_])

        pltpu.make_async_copy(
            table_hbm.at[idx_v[p]], rows_v[p], gsem[p]
        ).wait()
        @pl.when(b + 2 < BATCH)
        def _():
            pltpu.async_copy(
                idx_hbm.at[pl.ds(idx_off(b + 2), CHUNK)], idx_v[p], isem[p])

        add_pos(rows_v[p])
        pltpu.async_copy(rows_v[p], out_hbm.at[b, pl.ds(base, CHUNK), :],
                         wsem[p])

    def group_body(g, carry):
        for p in range(2):
            step(g * 2 + p, p)
        return carry

    lax.fori_loop(0, BATCH // 2, group_body, 0)

    for p in range(2):
        pltpu.make_async_copy(
            rows_v[p], out_hbm.at[0, pl.ds(base, CHUNK), :], wsem[p]
        ).wait()


def kernel(input_ids, emb_table, pos_enc):
    out = _emb_kernel(input_ids.astype(jnp.int32).reshape(BATCH * SEQ),
                      emb_table, pos_enc.reshape(SEQ, D))
    return out


# speedup vs baseline: 1.9751x; 1.1626x over previous
import functools

import jax
import jax.numpy as jnp
from jax import lax
from jax.experimental import pallas as pl
from jax.experimental.pallas import tpu as pltpu
from jax.experimental.pallas import tpu_sc as plsc

BATCH = 32
SEQ = 2048
D = 128
NUM_WORKERS = 32
CHUNK = SEQ // NUM_WORKERS
VECS = D // 16
GROUP = 4
ROWS = GROUP * CHUNK
IPG = 128
NGATH = ROWS // IPG
NSTEP = BATCH // GROUP


@functools.partial(
    pl.kernel,
    mesh=plsc.VectorSubcoreMesh(core_axis_name="c", subcore_axis_name="s"),
    out_type=jax.ShapeDtypeStruct((BATCH, SEQ, D), jnp.float32),
    scratch_types=[
        pltpu.VMEM((NGATH, IPG), jnp.int32),
        pltpu.VMEM((NGATH, IPG), jnp.int32),
        pltpu.VMEM((CHUNK, D), jnp.float32),
        pltpu.VMEM((ROWS, D), jnp.float32),
        pltpu.VMEM((ROWS, D), jnp.float32),
        pltpu.SemaphoreType.DMA,
        pltpu.SemaphoreType.DMA,
        pltpu.SemaphoreType.DMA,
        pltpu.SemaphoreType.DMA,
        pltpu.SemaphoreType.DMA,
        pltpu.SemaphoreType.DMA,
    ],
)
def _emb_kernel(idx_hbm, table_hbm, pos_hbm, out_hbm,
                idx0, idx1, pos_v, rows0, rows1,
                gsem0, gsem1, wsem0, wsem1, isem0, isem1):
    c = lax.axis_index("c")
    s = lax.axis_index("s")
    wid = s * 2 + c
    base = wid * CHUNK

    idx_v = (idx0, idx1)
    rows_v = (rows0, rows1)
    gsem = (gsem0, gsem1)
    wsem = (wsem0, wsem1)
    isem = (isem0, isem1)

    def load_idx(g, p, sync):
        for j in range(GROUP):
            off = pl.multiple_of((g * GROUP + j) * SEQ + base, CHUNK)
            dst = idx_v[p].at[j * CHUNK // IPG, pl.ds((j * CHUNK) % IPG, CHUNK)]
            if sync:
                pltpu.sync_copy(idx_hbm.at[pl.ds(off, CHUNK)], dst)
            else:
                pltpu.async_copy(idx_hbm.at[pl.ds(off, CHUNK)], dst, isem[p])

    def wait_idx(p):
        for j in range(GROUP):
            pltpu.make_async_copy(
                idx_hbm.at[pl.ds(0, CHUNK)],
                idx_v[p].at[0, pl.ds(0, CHUNK)], isem[p]).wait()

    def start_gathers(p):
        for k in range(NGATH):
            pltpu.async_copy(table_hbm.at[idx_v[p].at[k]],
                             rows_v[p].at[pl.ds(k * IPG, IPG), :], gsem[p])

    def wait_gathers(p):
        for k in range(NGATH):
            pltpu.make_async_copy(
                table_hbm.at[idx_v[p].at[k]],
                rows_v[p].at[pl.ds(k * IPG, IPG), :], gsem[p]).wait()

    def start_writes(g, p):
        for j in range(GROUP):
            pltpu.async_copy(
                rows_v[p].at[pl.ds(j * CHUNK, CHUNK), :],
                out_hbm.at[g * GROUP + j, pl.ds(base, CHUNK), :], wsem[p])

    def wait_writes(p):
        pltpu.make_async_copy(
            rows_v[p], out_hbm.at[0, pl.ds(0, SEQ), :].at[pl.ds(0, ROWS), :],
            wsem[p]).wait()

    def add_pos(p):
        rows = rows_v[p]

        def add_body(r2, carry):
            for u in range(2):
                r = r2 * 2 + u
                for cc in range(VECS):
                    sl = pl.ds(cc * 16, 16)
                    pv = pos_v[r, sl]
                    for j in range(GROUP):
                        plsc.addupdate(rows.at[j * CHUNK + r, sl], pv)
            return carry

        lax.fori_loop(0, CHUNK // 2, add_body, 0)

    pltpu.sync_copy(pos_hbm.at[pl.ds(base, CHUNK), :], pos_v)

    load_idx(0, 0, sync=True)
    start_gathers(0)
    load_idx(1, 1, sync=False)

    def step(g, p):
        np_ = 1 - p
        @pl.when(g + 1 < NSTEP)
        def _():
            wait_idx(np_)
            @pl.when(g >= 1)
            def _():
                wait_writes(np_)
            start_gathers(np_)

        wait_gathers(p)
        @pl.when(g + 2 < NSTEP)
        def _():
            load_idx(g + 2, p, sync=False)

        add_pos(p)
        start_writes(g, p)

    def group_body(h, carry):
        for p in range(2):
            step(h * 2 + p, p)
        return carry

    lax.fori_loop(0, NSTEP // 2, group_body, 0)

    for p in range(2):
        wait_writes(p)


def kernel(input_ids, emb_table, pos_enc):
    out = _emb_kernel(input_ids.astype(jnp.int32).reshape(BATCH * SEQ),
                      emb_table, pos_enc.reshape(SEQ, D))
    return out
